# baseline (speedup 1.0000x reference)
import functools

import jax
import jax.numpy as jnp
from jax import lax
from jax.experimental import pallas as pl
from jax.experimental.pallas import tpu as pltpu
from jax.experimental.pallas import tpu_sc as plsc

DIM = 32
LANES = 16
NUM_CORES = 2
NUM_SUBCORES = 16
NUM_WORKERS = NUM_CORES * NUM_SUBCORES


def _gmf_sc(u, i, u_table, i_table, w_flat, b_vec):
    batch = u.shape[0]
    bpw = batch // NUM_WORKERS

    mesh = plsc.VectorSubcoreMesh(core_axis_name="c", subcore_axis_name="s")

    @functools.partial(
        pl.kernel,
        mesh=mesh,
        compiler_params=pltpu.CompilerParams(
            needs_layout_passes=False, use_tc_tiling_on_sc=False),
        out_type=jax.ShapeDtypeStruct((batch,), jnp.float32),
        scratch_types=[
            pltpu.VMEM((bpw,), jnp.int32),
            pltpu.VMEM((bpw,), jnp.int32),
            pltpu.VMEM((bpw, DIM), jnp.float32),
            pltpu.VMEM((bpw, DIM), jnp.float32),
            pltpu.VMEM((bpw,), jnp.float32),
            pltpu.VMEM((DIM,), jnp.float32),
            pltpu.VMEM((LANES,), jnp.float32),
            pltpu.VMEM((LANES * LANES,), jnp.float32),
            pltpu.SemaphoreType.DMA,
            pltpu.SemaphoreType.DMA,
        ],
    )
    def k(u_hbm, i_hbm, ut_hbm, it_hbm, w_hbm, b_hbm, out_hbm,
          uidx_v, iidx_v, urows_v, irows_v, out_v, w_v, b_v, qt_v, sem_u, sem_i):
        wid = lax.axis_index("s") * NUM_CORES + lax.axis_index("c")
        base = wid * bpw

        pltpu.sync_copy(u_hbm.at[pl.ds(base, bpw)], uidx_v)
        pltpu.sync_copy(i_hbm.at[pl.ds(base, bpw)], iidx_v)
        pltpu.sync_copy(w_hbm, w_v)
        pltpu.sync_copy(b_hbm, b_v)

        cp_u = pltpu.async_copy(ut_hbm.at[uidx_v], urows_v, sem_u)
        cp_i = pltpu.async_copy(it_hbm.at[iidx_v], irows_v, sem_i)
        cp_u.wait()
        cp_i.wait()

        w0 = w_v[pl.ds(0, LANES)]
        w1 = w_v[pl.ds(LANES, LANES)]
        b_bcast = b_v[...]
        lane = lax.iota(jnp.int32, LANES)

        def group_body(g, carry):
            base_r = g * LANES
            for r in range(LANES):
                q = (urows_v[base_r + r, pl.ds(0, LANES)]
                     * irows_v[base_r + r, pl.ds(0, LANES)] * w0
                     + urows_v[base_r + r, pl.ds(LANES, LANES)]
                     * irows_v[base_r + r, pl.ds(LANES, LANES)] * w1)
                plsc.store_scatter(qt_v, [lane * LANES + r], q)
            acc = qt_v[pl.ds(0, LANES)]
            for c in range(1, LANES):
                acc = acc + qt_v[pl.ds(c * LANES, LANES)]
            x = acc + b_bcast
            out_v[pl.ds(base_r, LANES)] = 1.0 / (1.0 + jnp.exp(-x))
            return carry

        lax.fori_loop(0, bpw // LANES, group_body, 0)

        pltpu.sync_copy(out_v, out_hbm.at[pl.ds(base, bpw)])

    return k(u, i, u_table, i_table, w_flat, b_vec)


def kernel(u, i, u_table, i_table, W, b):
    w_flat = W.reshape(DIM)
    b_vec = jnp.broadcast_to(b.reshape(()), (LANES,))
    out = _gmf_sc(u, i, u_table, i_table, w_flat, b_vec)
    return out.reshape(-1, 1)


# speedup vs baseline: 3.3514x; 3.3514x over previous
import functools

import jax
import jax.numpy as jnp
from jax import lax
from jax.experimental import pallas as pl
from jax.experimental.pallas import tpu as pltpu
from jax.experimental.pallas import tpu_sc as plsc

DIM = 32
LANES = 16
NUM_CORES = 2
NUM_SUBCORES = 16
NUM_WORKERS = NUM_CORES * NUM_SUBCORES
SUBLANES = 8
TILE_ROWS = DIM // SUBLANES
GRP = 8


def _gmf_sc(u, i, ut3, it3, w_flat, b_vec):
    batch = u.shape[0]
    bpw = batch // NUM_WORKERS

    mesh = plsc.VectorSubcoreMesh(core_axis_name="c", subcore_axis_name="s")

    slot_ty = pltpu.VMEM((TILE_ROWS, SUBLANES, 128), jnp.float32)

    @functools.partial(
        pl.kernel,
        mesh=mesh,
        compiler_params=pltpu.CompilerParams(
            needs_layout_passes=False, use_tc_tiling_on_sc=True),
        out_type=jax.ShapeDtypeStruct((batch,), jnp.float32),
        scratch_types=(
            [pltpu.VMEM((bpw,), jnp.int32)] * 2
            + [slot_ty] * (2 * GRP)
            + [
                pltpu.VMEM((bpw,), jnp.float32),
                pltpu.VMEM((DIM,), jnp.float32),
                pltpu.VMEM((LANES,), jnp.float32),
                pltpu.SemaphoreType.DMA,
                pltpu.SemaphoreType.DMA,
            ]
        ),
    )
    def k(u_hbm, i_hbm, ut_hbm, it_hbm, w_hbm, b_hbm, out_hbm,
          uidx_v, iidx_v, *rest):
        uslots = rest[:GRP]
        islots = rest[GRP:2 * GRP]
        out_v, w_v, b_v, sem_u, sem_i = rest[2 * GRP:]

        wid = lax.axis_index("s") * NUM_CORES + lax.axis_index("c")
        base = wid * bpw

        pltpu.sync_copy(u_hbm.at[pl.ds(base, bpw)], uidx_v)
        pltpu.sync_copy(i_hbm.at[pl.ds(base, bpw)], iidx_v)
        pltpu.sync_copy(w_hbm, w_v)
        pltpu.sync_copy(b_hbm, b_v)

        w0 = w_v[pl.ds(0, LANES)]
        w1 = w_v[pl.ds(LANES, LANES)]
        b_bcast = b_v[...]
        lane = lax.iota(jnp.int32, LANES)
        g_lo = lane // SUBLANES
        s_lo = lane % SUBLANES
        g_hi = g_lo + 2
        s_hi = s_lo

        def extract(slot, co):
            cov = jnp.full((LANES,), co, jnp.int32)
            lo = plsc.load_gather(slot, [g_lo, s_lo, cov])
            hi = plsc.load_gather(slot, [g_hi, s_hi, cov])
            return lo, hi

        def group_body(grp, carry):
            off16 = grp * LANES
            acc = b_bcast
            for half in range(LANES // GRP):
                off = off16 + half * GRP
                uidx = uidx_v[pl.ds(off, LANES)]
                iidx = iidx_v[pl.ds(off, LANES)]
                uco = jnp.bitwise_and(uidx, 127)
                ico = jnp.bitwise_and(iidx, 127)
                ua = uidx - uco
                ia = iidx - ico
                copies = []
                for t in range(GRP):
                    a_u = pl.multiple_of(ua[t], 128)
                    a_i = pl.multiple_of(ia[t], 128)
                    copies.append(pltpu.async_copy(
                        ut_hbm.at[:, :, pl.ds(a_u, 128)], uslots[t], sem_u))
                    copies.append(pltpu.async_copy(
                        it_hbm.at[:, :, pl.ds(a_i, 128)], islots[t], sem_i))
                for cp in copies:
                    cp.wait()
                for t in range(GRP):
                    u0, u1 = extract(uslots[t], uco[t])
                    i0, i1 = extract(islots[t], ico[t])
                    q = u0 * i0 * w0 + u1 * i1 * w1
                    s = jnp.sum(q)
                    acc = jnp.where(lane == half * GRP + t, acc + s, acc)
            out_v[pl.ds(off16, LANES)] = 1.0 / (1.0 + jnp.exp(-acc))
            return carry

        lax.fori_loop(0, bpw // LANES, group_body, 0)

        pltpu.sync_copy(out_v, out_hbm.at[pl.ds(base, bpw)])

    return k(u, i, ut3, it3, w_flat, b_vec)


def kernel(u, i, u_table, i_table, W, b):
    n_user = u_table.shape[0]
    n_item = i_table.shape[0]
    ut3 = u_table.T.reshape(TILE_ROWS, SUBLANES, n_user)
    it3 = i_table.T.reshape(TILE_ROWS, SUBLANES, n_item)
    w_flat = W.reshape(DIM)
    b_vec = jnp.broadcast_to(b.reshape(()), (LANES,))
    out = _gmf_sc(u, i, ut3, it3, w_flat, b_vec)
    return out.reshape(-1, 1)


# speedup vs baseline: 3.7259x; 1.1118x over previous
import functools

import jax
import jax.numpy as jnp
from jax import lax
from jax.experimental import pallas as pl
from jax.experimental.pallas import tpu as pltpu
from jax.experimental.pallas import tpu_sc as plsc

DIM = 32
LANES = 16
NUM_CORES = 2
NUM_SUBCORES = 16
NUM_WORKERS = NUM_CORES * NUM_SUBCORES
SUBLANES = 8
TILE_ROWS = DIM // SUBLANES
SET = 4
SETS_PER_GROUP = LANES // SET


def _gmf_sc(u, i, ut3, it3, w_flat, b_vec):
    batch = u.shape[0]
    bpw = batch // NUM_WORKERS
    idx_len = bpw + LANES

    mesh = plsc.VectorSubcoreMesh(core_axis_name="c", subcore_axis_name="s")

    slot_ty = pltpu.VMEM((TILE_ROWS, SUBLANES, 128), jnp.float32)

    @functools.partial(
        pl.kernel,
        mesh=mesh,
        compiler_params=pltpu.CompilerParams(
            needs_layout_passes=False, use_tc_tiling_on_sc=True),
        out_type=jax.ShapeDtypeStruct((batch,), jnp.float32),
        scratch_types=(
            [pltpu.VMEM((idx_len,), jnp.int32)] * 2
            + [slot_ty] * (4 * SET)
            + [
                pltpu.VMEM((bpw,), jnp.float32),
                pltpu.VMEM((DIM,), jnp.float32),
                pltpu.VMEM((LANES,), jnp.float32),
            ]
            + [pltpu.SemaphoreType.DMA] * 4
        ),
    )
    def k(u_hbm, i_hbm, ut_hbm, it_hbm, w_hbm, b_hbm, out_hbm,
          uidx_v, iidx_v, *rest):
        uslots = (rest[0:SET], rest[2 * SET:3 * SET])
        islots = (rest[SET:2 * SET], rest[3 * SET:4 * SET])
        out_v, w_v, b_v = rest[4 * SET:4 * SET + 3]
        sems_u = (rest[4 * SET + 3], rest[4 * SET + 5])
        sems_i = (rest[4 * SET + 4], rest[4 * SET + 6])

        wid = lax.axis_index("s") * NUM_CORES + lax.axis_index("c")
        base = wid * bpw

        pltpu.sync_copy(u_hbm.at[pl.ds(base, bpw)], uidx_v.at[pl.ds(0, bpw)])
        pltpu.sync_copy(i_hbm.at[pl.ds(base, bpw)], iidx_v.at[pl.ds(0, bpw)])
        pltpu.sync_copy(w_hbm, w_v)
        pltpu.sync_copy(b_hbm, b_v)
        zeros16 = jnp.zeros((LANES,), jnp.int32)
        uidx_v[pl.ds(bpw, LANES)] = zeros16
        iidx_v[pl.ds(bpw, LANES)] = zeros16

        w0 = w_v[pl.ds(0, LANES)]
        w1 = w_v[pl.ds(LANES, LANES)]
        b_bcast = b_v[...]
        lane = lax.iota(jnp.int32, LANES)
        g_lo = lane // SUBLANES
        s_lo = lane % SUBLANES
        g_hi = g_lo + 2
        s_hi = s_lo

        def fetch_set(par, base_t):
            uidx = uidx_v[pl.ds(base_t, LANES)]
            iidx = iidx_v[pl.ds(base_t, LANES)]
            ua = uidx - jnp.bitwise_and(uidx, 127)
            ia = iidx - jnp.bitwise_and(iidx, 127)
            for t in range(SET):
                a_u = pl.multiple_of(ua[t], 128)
                a_i = pl.multiple_of(ia[t], 128)
                pltpu.async_copy(
                    ut_hbm.at[:, :, pl.ds(a_u, 128)], uslots[par][t],
                    sems_u[par])
                pltpu.async_copy(
                    it_hbm.at[:, :, pl.ds(a_i, 128)], islots[par][t],
                    sems_i[par])

        def wait_set(par):
            for t in range(SET):
                pltpu.make_async_copy(
                    ut_hbm.at[:, :, pl.ds(0, 128)], uslots[par][t],
                    sems_u[par]).wait()
                pltpu.make_async_copy(
                    it_hbm.at[:, :, pl.ds(0, 128)], islots[par][t],
                    sems_i[par]).wait()

        def extract(slot, co):
            cov = jnp.full((LANES,), co, jnp.int32)
            lo = plsc.load_gather(slot, [g_lo, s_lo, cov])
            hi = plsc.load_gather(slot, [g_hi, s_hi, cov])
            return lo, hi

        def process_set(par, base_t, lane_base, acc):
            uidx = uidx_v[pl.ds(base_t, LANES)]
            iidx = iidx_v[pl.ds(base_t, LANES)]
            uco = jnp.bitwise_and(uidx, 127)
            ico = jnp.bitwise_and(iidx, 127)
            for t in range(SET):
                u0, u1 = extract(uslots[par][t], uco[t])
                i0, i1 = extract(islots[par][t], ico[t])
                q = u0 * i0 * w0 + u1 * i1 * w1
                s = jnp.sum(q)
                acc = jnp.where(lane == lane_base + t, acc + s, acc)
            return acc

        fetch_set(0, 0)

        def group_body(grp, carry):
            off16 = grp * LANES
            acc = b_bcast
            for s in range(SETS_PER_GROUP):
                fetch_set((s + 1) % 2, off16 + (s + 1) * SET)
                wait_set(s % 2)
                acc = process_set(s % 2, off16 + s * SET, s * SET, acc)
            out_v[pl.ds(off16, LANES)] = 1.0 / (1.0 + jnp.exp(-acc))
            return carry

        lax.fori_loop(0, bpw // LANES, group_body, 0)
        wait_set(0)

        pltpu.sync_copy(out_v, out_hbm.at[pl.ds(base, bpw)])

    return k(u, i, ut3, it3, w_flat, b_vec)


def kernel(u, i, u_table, i_table, W, b):
    n_user = u_table.shape[0]
    n_item = i_table.shape[0]
    ut3 = u_table.T.reshape(TILE_ROWS, SUBLANES, n_user)
    it3 = i_table.T.reshape(TILE_ROWS, SUBLANES, n_item)
    w_flat = W.reshape(DIM)
    b_vec = jnp.broadcast_to(b.reshape(()), (LANES,))
    out = _gmf_sc(u, i, ut3, it3, w_flat, b_vec)
    return out.reshape(-1, 1)


# speedup vs baseline: 3.7578x; 1.0086x over previous
import functools

import jax
import jax.numpy as jnp
from jax import lax
from jax.experimental import pallas as pl
from jax.experimental.pallas import tpu as pltpu
from jax.experimental.pallas import tpu_sc as plsc

DIM = 32
LANES = 16
NUM_CORES = 2
NUM_SUBCORES = 16
NUM_WORKERS = NUM_CORES * NUM_SUBCORES
SUBLANES = 8
TILE_ROWS = DIM // SUBLANES
SET = 4
SETS_PER_GROUP = LANES // SET


def _gmf_sc(u, i, ut3, it3, w_flat, b_vec):
    batch = u.shape[0]
    bpw = batch // NUM_WORKERS
    idx_len = bpw + LANES

    mesh = plsc.VectorSubcoreMesh(core_axis_name="c", subcore_axis_name="s")

    slot_ty = pltpu.VMEM((TILE_ROWS, SUBLANES, 128), jnp.float32)

    @functools.partial(
        pl.kernel,
        mesh=mesh,
        compiler_params=pltpu.CompilerParams(
            needs_layout_passes=False, use_tc_tiling_on_sc=True),
        out_type=jax.ShapeDtypeStruct((batch,), jnp.float32),
        scratch_types=(
            [pltpu.VMEM((idx_len,), jnp.int32)] * 2
            + [slot_ty] * (4 * SET)
            + [
                pltpu.VMEM((bpw,), jnp.float32),
                pltpu.VMEM((DIM,), jnp.float32),
                pltpu.VMEM((LANES,), jnp.float32),
            ]
            + [pltpu.SemaphoreType.DMA] * 4
        ),
    )
    def k(u_hbm, i_hbm, ut_hbm, it_hbm, w_hbm, b_hbm, out_hbm,
          uidx_v, iidx_v, *rest):
        uslots = (rest[0:SET], rest[2 * SET:3 * SET])
        islots = (rest[SET:2 * SET], rest[3 * SET:4 * SET])
        out_v, w_v, b_v = rest[4 * SET:4 * SET + 3]
        sems_u = (rest[4 * SET + 3], rest[4 * SET + 5])
        sems_i = (rest[4 * SET + 4], rest[4 * SET + 6])

        wid = lax.axis_index("s") * NUM_CORES + lax.axis_index("c")
        base = wid * bpw

        pltpu.sync_copy(u_hbm.at[pl.ds(base, bpw)], uidx_v.at[pl.ds(0, bpw)])
        pltpu.sync_copy(i_hbm.at[pl.ds(base, bpw)], iidx_v.at[pl.ds(0, bpw)])
        pltpu.sync_copy(w_hbm, w_v)
        pltpu.sync_copy(b_hbm, b_v)
        zeros16 = jnp.zeros((LANES,), jnp.int32)
        uidx_v[pl.ds(bpw, LANES)] = zeros16
        iidx_v[pl.ds(bpw, LANES)] = zeros16

        w0 = w_v[pl.ds(0, LANES)]
        w1 = w_v[pl.ds(LANES, LANES)]
        b_bcast = b_v[...]
        lane = lax.iota(jnp.int32, LANES)
        g_lo = lane // SUBLANES
        s_lo = lane % SUBLANES
        g_hi = g_lo + 2
        s_hi = s_lo

        def fetch_set(par, base_t):
            uidx = uidx_v[pl.ds(base_t, LANES)]
            iidx = iidx_v[pl.ds(base_t, LANES)]
            ua = uidx - jnp.bitwise_and(uidx, 127)
            ia = iidx - jnp.bitwise_and(iidx, 127)
            for t in range(SET):
                a_u = pl.multiple_of(ua[t], 128)
                a_i = pl.multiple_of(ia[t], 128)
                for g in range(TILE_ROWS):
                    pltpu.async_copy(
                        ut_hbm.at[g, :, pl.ds(a_u, 128)],
                        uslots[par][t].at[g], sems_u[par])
                    pltpu.async_copy(
                        it_hbm.at[g, :, pl.ds(a_i, 128)],
                        islots[par][t].at[g], sems_i[par])

        def wait_set(par):
            for t in range(SET):
                pltpu.make_async_copy(
                    ut_hbm.at[:, :, pl.ds(0, 128)], uslots[par][t],
                    sems_u[par]).wait()
                pltpu.make_async_copy(
                    it_hbm.at[:, :, pl.ds(0, 128)], islots[par][t],
                    sems_i[par]).wait()

        def extract(slot, co):
            cov = jnp.full((LANES,), co, jnp.int32)
            lo = plsc.load_gather(slot, [g_lo, s_lo, cov])
            hi = plsc.load_gather(slot, [g_hi, s_hi, cov])
            return lo, hi

        def process_set(par, base_t, lane_base, acc):
            uidx = uidx_v[pl.ds(base_t, LANES)]
            iidx = iidx_v[pl.ds(base_t, LANES)]
            uco = jnp.bitwise_and(uidx, 127)
            ico = jnp.bitwise_and(iidx, 127)
            for t in range(SET):
                u0, u1 = extract(uslots[par][t], uco[t])
                i0, i1 = extract(islots[par][t], ico[t])
                q = u0 * i0 * w0 + u1 * i1 * w1
                s = jnp.sum(q)
                acc = jnp.where(lane == lane_base + t, acc + s, acc)
            return acc

        fetch_set(0, 0)

        def group_body(grp, carry):
            off16 = grp * LANES
            acc = b_bcast
            for s in range(SETS_PER_GROUP):
                fetch_set((s + 1) % 2, off16 + (s + 1) * SET)
                wait_set(s % 2)
                acc = process_set(s % 2, off16 + s * SET, s * SET, acc)
            out_v[pl.ds(off16, LANES)] = 1.0 / (1.0 + jnp.exp(-acc))
            return carry

        lax.fori_loop(0, bpw // LANES, group_body, 0)
        wait_set(0)

        pltpu.sync_copy(out_v, out_hbm.at[pl.ds(base, bpw)])

    return k(u, i, ut3, it3, w_flat, b_vec)


def kernel(u, i, u_table, i_table, W, b):
    n_user = u_table.shape[0]
    n_item = i_table.shape[0]
    ut3 = u_table.T.reshape(TILE_ROWS, SUBLANES, n_user)
    it3 = i_table.T.reshape(TILE_ROWS, SUBLANES, n_item)
    w_flat = W.reshape(DIM)
    b_vec = jnp.broadcast_to(b.reshape(()), (LANES,))
    out = _gmf_sc(u, i, ut3, it3, w_flat, b_vec)
    return out.reshape(-1, 1)
